# Initial kernel scaffold; baseline (speedup 1.0000x reference)
import jax
import jax.numpy as jnp
from jax.experimental import pallas as pl


def kernel(flat, cu_seqlens):
    raise NotImplementedError("write your pallas kernel here")


# speedup vs baseline: 2.5317x; 2.5317x over previous
import jax
import jax.numpy as jnp
from jax import lax
from jax.experimental import pallas as pl
from jax.experimental.pallas import tpu as pltpu, tpu_sc as plsc

B = 16
MAX_LEN = 2048
D = 512
TOTAL = 16384
NW = 32
ROWS_PER_W = (B * MAX_LEN) // NW
C = 64
NCHUNK = ROWS_PER_W // C
LANES = 16


def _unrag_body(cu_hbm, flat_hbm, out_hbm, cu_v, buf, zbuf):
    cid = lax.axis_index("c")
    sid = lax.axis_index("s")
    wid = sid * 2 + cid
    b = wid // 2
    j0 = (wid % 2) * (MAX_LEN // 2)
    obase = wid * ROWS_PER_W

    pltpu.sync_copy(cu_hbm, cu_v)
    seg_start = cu_v[pl.ds(b, LANES)][0]
    seg_end = cu_v[pl.ds(b + 1, LANES)][0]

    start = seg_start + j0
    n = jnp.clip(seg_end - start, 0, ROWS_PER_W)
    nfull = n // C
    rem = n - nfull * C

    zeros16 = jnp.zeros((LANES,), jnp.float32)

    def zrow(i, carry):
        for k in range(D // LANES):
            zbuf[i, pl.ds(k * LANES, LANES)] = zeros16
        return carry

    lax.fori_loop(0, C, zrow, 0)

    def full_chunk(c, carry):
        pltpu.sync_copy(flat_hbm.at[pl.ds(start + c * C, C)], buf)
        pltpu.sync_copy(buf, out_hbm.at[pl.ds(obase + c * C, C)])
        return carry

    lax.fori_loop(0, nfull, full_chunk, 0)

    first_zero = nfull + (rem > 0).astype(jnp.int32)

    def zero_chunk(c, carry):
        pltpu.sync_copy(zbuf, out_hbm.at[pl.ds(obase + c * C, C)])
        return carry

    lax.fori_loop(first_zero, NCHUNK, zero_chunk, 0)

    @pl.when(rem > 0)
    def _partial():
        s_e = jnp.clip(seg_end - C, 0, TOTAL - C)
        delta = (start + nfull * C) - s_e
        pltpu.sync_copy(flat_hbm.at[pl.ds(s_e, C)], buf)

        def move(i, carry):
            for k in range(D // LANES):
                zbuf[i, pl.ds(k * LANES, LANES)] = buf[delta + i, pl.ds(k * LANES, LANES)]
            return carry

        lax.fori_loop(0, rem, move, 0)
        pltpu.sync_copy(zbuf, out_hbm.at[pl.ds(obase + nfull * C, C)])


def kernel(flat, cu_seqlens):
    cu = cu_seqlens.astype(jnp.int32)
    cu = cu.at[0].set(0).at[-1].set(TOTAL)
    cu_pad = jnp.zeros((2 * LANES,), jnp.int32).at[:B + 1].set(cu)

    run = pl.kernel(
        _unrag_body,
        out_type=jax.ShapeDtypeStruct((B * MAX_LEN, D), jnp.float32),
        mesh=plsc.VectorSubcoreMesh(core_axis_name="c", subcore_axis_name="s"),
        compiler_params=pltpu.CompilerParams(use_tc_tiling_on_sc=False),
        scratch_types=[
            pltpu.VMEM((2 * LANES,), jnp.int32),
            pltpu.VMEM((C, D), jnp.float32),
            pltpu.VMEM((C, D), jnp.float32),
        ],
    )
    dense = run(cu_pad, flat)
    return dense.reshape(B, MAX_LEN, D)


# speedup vs baseline: 2.7609x; 1.0905x over previous
import jax
import jax.numpy as jnp
from jax import lax
from jax.experimental import pallas as pl
from jax.experimental.pallas import tpu as pltpu, tpu_sc as plsc

B = 16
MAX_LEN = 2048
D = 512
TOTAL = 16384
NW = 32
ROWS_PER_W = (B * MAX_LEN) // NW
C = 64
NCHUNK = ROWS_PER_W // C
LANES = 16


def _unrag_body(cu_hbm, flat_hbm, out_hbm, cu_v, buf, zbuf, rsem, wsem, zsem):
    cid = lax.axis_index("c")
    sid = lax.axis_index("s")
    wid = sid * 2 + cid
    b = wid // 2
    j0 = (wid % 2) * (MAX_LEN // 2)
    obase = wid * ROWS_PER_W

    pltpu.sync_copy(cu_hbm, cu_v)
    seg_start = cu_v[pl.ds(b, LANES)][0]
    seg_end = cu_v[pl.ds(b + 1, LANES)][0]

    start = seg_start + j0
    n = jnp.clip(seg_end - start, 0, ROWS_PER_W)
    nfull = n // C
    rem = n - nfull * C

    zeros16 = jnp.zeros((LANES,), jnp.float32)

    def zrow(i, carry):
        for k in range(D // LANES):
            zbuf[i, pl.ds(k * LANES, LANES)] = zeros16
        return carry

    lax.fori_loop(0, C, zrow, 0)

    first_zero = nfull + (rem > 0).astype(jnp.int32)
    nzero = NCHUNK - first_zero

    def zero_chunk(c, carry):
        pltpu.make_async_copy(zbuf, out_hbm.at[pl.ds(obase + c * C, C)], zsem).start()
        return carry

    lax.fori_loop(first_zero, NCHUNK, zero_chunk, 0)

    def read_chunk(c):
        ph = lax.rem(c, 2)
        pltpu.make_async_copy(flat_hbm.at[pl.ds(start + c * C, C)],
                              buf.at[ph], rsem).start()

    @pl.when(nfull > 0)
    def _prime():
        read_chunk(0)

    def full_chunk(c, carry):
        ph = lax.rem(c, 2)
        pltpu.make_async_copy(flat_hbm.at[pl.ds(start + c * C, C)],
                              buf.at[ph], rsem).wait()

        @pl.when(c + 1 < nfull)
        def _next():
            read_chunk(c + 1)

        pltpu.make_async_copy(buf.at[ph], out_hbm.at[pl.ds(obase + c * C, C)],
                              wsem).start()
        pltpu.make_async_copy(buf.at[ph], out_hbm.at[pl.ds(obase + c * C, C)],
                              wsem).wait()
        return carry

    lax.fori_loop(0, nfull, full_chunk, 0)

    def zero_drain(c, carry):
        pltpu.make_async_copy(zbuf, out_hbm.at[pl.ds(obase + c * C, C)], zsem).wait()
        return carry

    lax.fori_loop(first_zero, NCHUNK, zero_drain, 0)

    @pl.when(rem > 0)
    def _partial():
        s_e = jnp.clip(seg_end - C, 0, TOTAL - C)
        delta = (start + nfull * C) - s_e
        pltpu.sync_copy(flat_hbm.at[pl.ds(s_e, C)], buf.at[0])

        def move(i, carry):
            for k in range(D // LANES):
                zbuf[i, pl.ds(k * LANES, LANES)] = buf[0, delta + i, pl.ds(k * LANES, LANES)]
            return carry

        lax.fori_loop(0, rem, move, 0)
        pltpu.sync_copy(zbuf, out_hbm.at[pl.ds(obase + nfull * C, C)])


def kernel(flat, cu_seqlens):
    cu = cu_seqlens.astype(jnp.int32)
    cu = cu.at[0].set(0).at[-1].set(TOTAL)
    cu_pad = jnp.zeros((2 * LANES,), jnp.int32).at[:B + 1].set(cu)

    run = pl.kernel(
        _unrag_body,
        out_type=jax.ShapeDtypeStruct((B * MAX_LEN, D), jnp.float32),
        mesh=plsc.VectorSubcoreMesh(core_axis_name="c", subcore_axis_name="s"),
        compiler_params=pltpu.CompilerParams(use_tc_tiling_on_sc=False),
        scratch_types=[
            pltpu.VMEM((2 * LANES,), jnp.int32),
            pltpu.VMEM((2, C, D), jnp.float32),
            pltpu.VMEM((C, D), jnp.float32),
            pltpu.SemaphoreType.DMA,
            pltpu.SemaphoreType.DMA,
            pltpu.SemaphoreType.DMA,
        ],
    )
    dense = run(cu_pad, flat)
    return dense.reshape(B, MAX_LEN, D)


# speedup vs baseline: 5.2282x; 1.8937x over previous
import jax
import jax.numpy as jnp
from jax import lax
from jax.experimental import pallas as pl
from jax.experimental.pallas import tpu as pltpu, tpu_sc as plsc

B = 16
MAX_LEN = 2048
D = 512
TOTAL = 16384
NW = 32
ROWS_PER_W = (B * MAX_LEN) // NW
C = 64
NCHUNK = ROWS_PER_W // C
LANES = 16


def _unrag_body(cu_hbm, flat_hbm, zeros_hbm, out_hbm,
                cu_v, buf, zbuf, gsem, wsem, zsem):
    cid = lax.axis_index("c")
    sid = lax.axis_index("s")
    wid = sid * 2 + cid
    b = wid // 2
    j0 = (wid % 2) * (MAX_LEN // 2)
    obase = wid * ROWS_PER_W

    pltpu.sync_copy(cu_hbm, cu_v)
    seg_start = cu_v[pl.ds(b, LANES)][0]
    seg_end = cu_v[pl.ds(b + 1, LANES)][0]

    start = seg_start + j0
    n = jnp.clip(seg_end - start, 0, ROWS_PER_W)
    nfull = n // C
    rem = n - nfull * C
    nceil = nfull + (rem > 0).astype(jnp.int32)

    pltpu.sync_copy(zeros_hbm, zbuf)

    def zero_chunk(c, carry):
        pltpu.make_async_copy(zbuf, out_hbm.at[pl.ds(obase + c * C, C)], zsem).start()
        return carry

    lax.fori_loop(nceil, NCHUNK, zero_chunk, 0)

    lane = lax.iota(jnp.int32, LANES)
    hi = seg_end - 1

    def gather_chunk(c):
        ph = lax.rem(c, 2)
        for q in range(C // LANES):
            idx = jnp.minimum(start + c * C + q * LANES + lane, hi)
            pltpu.make_async_copy(flat_hbm.at[idx],
                                  buf.at[ph, pl.ds(q * LANES, LANES)],
                                  gsem).start()

    @pl.when(nceil > 0)
    def _prime():
        gather_chunk(0)

    def data_chunk(c, carry):
        ph = lax.rem(c, 2)
        for q in range(C // LANES):
            idx = jnp.minimum(start + c * C + q * LANES + lane, hi)
            pltpu.make_async_copy(flat_hbm.at[idx],
                                  buf.at[ph, pl.ds(q * LANES, LANES)],
                                  gsem).wait()

        @pl.when(c + 1 < nceil)
        def _next():
            gather_chunk(c + 1)

        @pl.when((c + 1 == nceil) & (rem > 0))
        def _zero_tail():
            zrow = jnp.zeros((LANES,), jnp.float32)

            def ztail(i, carry):
                for k in range(D // LANES):
                    buf[ph, i, pl.ds(k * LANES, LANES)] = zrow
                return carry

            lax.fori_loop(rem, C, ztail, 0)

        pltpu.make_async_copy(buf.at[ph], out_hbm.at[pl.ds(obase + c * C, C)],
                              wsem).start()
        pltpu.make_async_copy(buf.at[ph], out_hbm.at[pl.ds(obase + c * C, C)],
                              wsem).wait()
        return carry

    lax.fori_loop(0, nceil, data_chunk, 0)

    def zero_drain(c, carry):
        pltpu.make_async_copy(zbuf, out_hbm.at[pl.ds(obase + c * C, C)], zsem).wait()
        return carry

    lax.fori_loop(nceil, NCHUNK, zero_drain, 0)


def kernel(flat, cu_seqlens):
    cu = cu_seqlens.astype(jnp.int32)
    cu = cu.at[0].set(0).at[-1].set(TOTAL)
    cu_pad = jnp.zeros((2 * LANES,), jnp.int32).at[:B + 1].set(cu)
    zeros_blk = jnp.zeros((C, D), jnp.float32)

    run = pl.kernel(
        _unrag_body,
        out_type=jax.ShapeDtypeStruct((B * MAX_LEN, D), jnp.float32),
        mesh=plsc.VectorSubcoreMesh(core_axis_name="c", subcore_axis_name="s"),
        scratch_types=[
            pltpu.VMEM((2 * LANES,), jnp.int32),
            pltpu.VMEM((2, C, D), jnp.float32),
            pltpu.VMEM((C, D), jnp.float32),
            pltpu.SemaphoreType.DMA,
            pltpu.SemaphoreType.DMA,
            pltpu.SemaphoreType.DMA,
        ],
    )
    dense = run(cu_pad, flat, zeros_blk)
    return dense.reshape(B, MAX_LEN, D)
